# baseline (speedup 1.0000x reference)
import functools

import jax
import jax.numpy as jnp
from jax.experimental import pallas as pl
from jax.experimental.pallas import tpu as pltpu
from jax.experimental.pallas import tpu_sc as plsc

N1, N2, C = 2500, 10000, 128
K_UP, K_G = 8, 16
R = 400
NB = N2 // R
EB = R * K_G
N_EDGE = N2 * (K_G + 1)
_BN_EPS = 1e-5


def _rep16(x):
    r, d = x.shape
    return jnp.broadcast_to(x[:, None, :], (r, K_G, d)).reshape(r * K_G, d)


def _sum8(x):
    n, d = x.shape
    return jnp.sum(x.reshape(n // 8, 8, d), axis=0)


def _edge_kernel(ag_ref, adst_ref, asrc_ref, v_ref, xint_ref, u_ref,
                 bp1_ref, gp_ref, btp_ref, wp2_ref, bp2_ref,
                 wa1_ref, ba1_ref, ga_ref, bta_ref, wa2_ref, ba2_ref,
                 wup_ref, bup_ref, out_ref,
                 sp1, sp2, sa1, sa2):
    p = pl.program_id(0)

    @pl.when((p == 0) & (pl.program_id(1) == 0))
    def _():
        sp1[...] = jnp.zeros_like(sp1)
        sp2[...] = jnp.zeros_like(sp2)
        sa1[...] = jnp.zeros_like(sa1)
        sa2[...] = jnp.zeros_like(sa2)

    f32 = jnp.float32
    dot = functools.partial(jnp.dot, preferred_element_type=f32)

    hp = (_rep16(u_ref[...]) + bp1_ref[...]) - ag_ref[:, 2 * C:]

    @pl.when(p == 0)
    def _():
        sp1[...] += _sum8(hp)
        sp2[...] += _sum8(hp * hp)
        out_ref[...] = jnp.zeros_like(out_ref)

    @pl.when(p > 0)
    def _():
        bp1v = bp1_ref[...]
        n_e = f32(N_EDGE)
        mu_p = (jnp.sum(sp1[...], 0, keepdims=True) + N2 * bp1v) / n_e
        msq_p = (jnp.sum(sp2[...], 0, keepdims=True) + N2 * bp1v * bp1v) / n_e
        inv_p = jax.lax.rsqrt(msq_p - mu_p * mu_p + _BN_EPS)

        def bn_p(h):
            return (h - mu_p) * inv_p * gp_ref[...] + btp_ref[...]

        delta = dot(jax.nn.relu(bn_p(hp)), wp2_ref[...].T) + bp2_ref[...]
        d0 = dot(jax.nn.relu(bn_p(bp1v)), wp2_ref[...].T) + bp2_ref[...]

        adst = adst_ref[...]
        alpha_in = _rep16(adst) - ag_ref[:, :C] + delta
        ha = dot(alpha_in, wa1_ref[...].T) + ba1_ref[...]
        ha_s = dot(adst - asrc_ref[...] + d0, wa1_ref[...].T) + ba1_ref[...]

        @pl.when(p == 1)
        def _():
            sa1[...] += _sum8(ha) + _sum8(ha_s)
            sa2[...] += _sum8(ha * ha) + _sum8(ha_s * ha_s)
            out_ref[...] = jnp.zeros_like(out_ref)

        @pl.when(p == 2)
        def _():
            mu_a = jnp.sum(sa1[...], 0, keepdims=True) / n_e
            msq_a = jnp.sum(sa2[...], 0, keepdims=True) / n_e
            inv_a = jax.lax.rsqrt(msq_a - mu_a * mu_a + _BN_EPS)

            def bn_a(h):
                return (h - mu_a) * inv_a * ga_ref[...] + bta_ref[...]

            al = dot(jax.nn.relu(bn_a(ha)), wa2_ref[...].T) + ba2_ref[...]
            al_s = dot(jax.nn.relu(bn_a(ha_s)), wa2_ref[...].T) + ba2_ref[...]

            al3 = al.reshape(R, K_G, C)
            m = jnp.maximum(jnp.max(al3, axis=1), al_s)
            ex = jnp.exp(al3 - m[:, None, :])
            ex_s = jnp.exp(al_s - m)
            den = jnp.sum(ex, axis=1) + ex_s + 1e-16
            vpd = (ag_ref[:, C:2 * C] + delta).reshape(R, K_G, C)
            num = jnp.sum(ex * vpd, axis=1) + ex_s * (v_ref[...] + d0)
            agg = num / den
            out_ref[...] = (dot(agg, wup_ref[...].T) + bup_ref[...]
                            + xint_ref[...])


def _edge_stage(ag, a_dst, a_src, v, x_int, u, bp1, gp, btp,
                wp2, bp2, wa1, ba1, ga, bta, wa2, ba2, w_up, b_up):
    blk = lambda r, c: pl.BlockSpec((r, c), lambda p, b: (b, 0))
    full = lambda r, c: pl.BlockSpec((r, c), lambda p, b: (0, 0))
    return pl.pallas_call(
        _edge_kernel,
        grid=(3, NB),
        in_specs=[
            blk(EB, 3 * C),
            blk(R, C), blk(R, C), blk(R, C), blk(R, C), blk(R, C),
            full(1, C), full(1, C), full(1, C),
            full(C, C), full(1, C),
            full(C, C), full(1, C), full(1, C), full(1, C),
            full(C, C), full(1, C),
            full(C, C), full(1, C),
        ],
        out_specs=blk(R, C),
        out_shape=jax.ShapeDtypeStruct((N2, C), jnp.float32),
        scratch_shapes=[pltpu.VMEM((8, C), jnp.float32)] * 4,
    )(ag, a_dst, a_src, v, x_int, u, bp1, gp, btp,
      wp2, bp2, wa1, ba1, ga, bta, wa2, ba2, w_up, b_up)


_BIG = 1e30
N1P = 2560
N2P = 10240


def _interp_kernel(x1_ref, pt1_ref, q_ref, wl1_ref, bl1_ref,
                   wv_ref, ws_ref, wd_ref, wp1q_ref,
                   xint_ref, v_ref, asrc_ref, adst_ref, u_ref, h1_ref):
    f32 = jnp.float32
    dot = functools.partial(jnp.dot, preferred_element_type=f32,
                            precision=jax.lax.Precision.HIGHEST)

    @pl.when(pl.program_id(0) == 0)
    def _():
        h1_ref[...] = dot(x1_ref[...], wl1_ref[...].T) + bl1_ref[...]

    d2 = _dist2(q_ref, pt1_ref)
    score = d2
    sel = jnp.zeros_like(score, dtype=jnp.bool_)
    cols = jax.lax.broadcasted_iota(jnp.int32, (1, N1P), 1)
    for _ in range(K_UP):
        val = jnp.min(score, axis=1, keepdims=True)
        eq = score == val
        idxc = jnp.where(eq, cols, jnp.int32(2 ** 30))
        si = jnp.min(idxc, axis=1, keepdims=True)
        hit = cols == si
        sel = sel | hit
        score = jnp.where(hit, _BIG, score)
    s = jnp.where(sel, 1.0 / jnp.clip(d2, 1e-16, None), 0.0)
    x_int = dot(s, h1_ref[...]) / jnp.sum(s, axis=1, keepdims=True)
    xint_ref[...] = x_int
    v_ref[...] = dot(x_int, wv_ref[...].T)
    asrc_ref[...] = dot(x_int, ws_ref[...].T)
    adst_ref[...] = dot(x_int, wd_ref[...].T)
    u_ref[...] = dot(q_ref[...], wp1q_ref[...].T)


def _interp_stage(x1p, pt1, pos2q, W_l1, b_l1, W_val, W_src, W_dst, Wp1q):
    blk = pl.BlockSpec((R, C), lambda b: (b, 0))
    full = lambda r, c: pl.BlockSpec((r, c), lambda b: (0, 0))
    o = jax.ShapeDtypeStruct((N2, C), jnp.float32)
    return pl.pallas_call(
        _interp_kernel,
        grid=(NB,),
        in_specs=[full(N1P, C), full(8, N1P), pl.BlockSpec((R, 8), lambda b: (b, 0)),
                  full(C, C), full(1, C), full(C, C), full(C, C), full(C, C),
                  full(C, 8)],
        out_specs=[blk] * 5,
        out_shape=[o, o, o, o, o],
        scratch_shapes=[pltpu.VMEM((N1P, C), jnp.float32)],
    )(x1p, pt1, pos2q, W_l1, b_l1, W_val, W_src, W_dst, Wp1q)


def _dist2(q_ref, pt_ref):
    dx = q_ref[:, 0:1] - pt_ref[0:1, :]
    dy = q_ref[:, 1:2] - pt_ref[1:2, :]
    dz = q_ref[:, 2:3] - pt_ref[2:3, :]
    return (dx * dx + dy * dy) + dz * dz + pt_ref[3:4, :]


RK = 80
_NL = 5
_NG = N2P // 128


def _extract16(pt2_ref, q_ref, b, cols, idx_ref):
    rows = jax.lax.broadcasted_iota(jnp.int32, (RK, 1), 0)
    score = _dist2(q_ref, pt2_ref)
    score = jnp.where(cols == b * RK + rows, _BIG, score)
    lane16 = jax.lax.broadcasted_iota(jnp.int32, (1, K_G), 1)
    idx_out = jnp.zeros((RK, K_G), jnp.int32)
    for k in range(K_G):
        val = jnp.min(score, axis=1, keepdims=True)
        eq = score == val
        idxc = jnp.where(eq, cols, jnp.int32(2 ** 30))
        si = jnp.min(idxc, axis=1, keepdims=True)
        score = jnp.where(idxc == si, _BIG, score)
        idx_out = jnp.where(lane16 == k, si, idx_out)
    idx_ref[...] = idx_out


def _knn16_kernel(pt2_ref, q_ref, idx_ref):
    b = pl.program_id(0)
    score = _dist2(q_ref, pt2_ref)
    cols = jax.lax.broadcasted_iota(jnp.int32, (1, N2P), 1)
    rows = jax.lax.broadcasted_iota(jnp.int32, (RK, 1), 0)
    score = jnp.where(cols == b * RK + rows, _BIG, score)

    s3 = score.reshape(RK, _NG, 128)
    li = jax.lax.broadcasted_iota(jnp.int32, (1, 1, 128), 2)
    cand_v, cand_i = [], []
    for _ in range(_NL):
        gv = jnp.min(s3, axis=2, keepdims=True)
        eq = s3 == gv
        larg = jnp.where(eq, li, jnp.int32(2 ** 30))
        lmin = jnp.min(larg, axis=2, keepdims=True)
        s3 = jnp.where(larg == lmin, _BIG, s3)
        cand_v.append(gv.reshape(RK, _NG))
        cand_i.append(lmin.reshape(RK, _NG))
    cv = jnp.concatenate(cand_v, axis=1)
    gidx = jax.lax.broadcasted_iota(jnp.int32, (1, _NL * _NG), 1) % _NG
    ci = jnp.concatenate(cand_i, axis=1) + gidx * 128

    lane16 = jax.lax.broadcasted_iota(jnp.int32, (1, K_G), 1)
    idx_out = jnp.zeros((RK, K_G), jnp.int32)
    used = jnp.zeros_like(cv)
    for k in range(K_G):
        val = jnp.min(cv, axis=1, keepdims=True)
        eq = cv == val
        idxc = jnp.where(eq, ci, jnp.int32(2 ** 30))
        si = jnp.min(idxc, axis=1, keepdims=True)
        hit = eq & (ci == si)
        used = used + jnp.where(hit, 1.0, 0.0)
        cv = jnp.where(hit, _BIG, cv)
        idx_out = jnp.where(lane16 == k, si, idx_out)
    per_group = jnp.sum(used.reshape(RK, _NL, _NG), axis=1)
    ok = jnp.max(per_group) < float(_NL)

    @pl.when(ok)
    def _():
        idx_ref[...] = idx_out

    @pl.when(jnp.logical_not(ok))
    def _():
        _extract16(pt2_ref, q_ref, b, cols, idx_ref)


def _knn16_stage(pt2, pos2q):
    return pl.pallas_call(
        _knn16_kernel,
        grid=(N2 // RK,),
        in_specs=[pl.BlockSpec((8, N2P), lambda b: (0, 0)),
                  pl.BlockSpec((RK, 8), lambda b: (b, 0))],
        out_specs=pl.BlockSpec((RK, K_G), lambda b: (b, 0)),
        out_shape=jax.ShapeDtypeStruct((N2, K_G), jnp.int32),
    )(pt2, pos2q)


_GW = 128
N_E16 = N2 * K_G


def _sc_gather(table, idx_flat):
    f32 = jnp.float32
    mesh = plsc.VectorSubcoreMesh(core_axis_name="c", subcore_axis_name="s")

    @functools.partial(
        pl.kernel,
        out_type=jax.ShapeDtypeStruct((N_E16, 3 * C), f32),
        mesh=mesh)
    def gather_kernel(tbl_hbm, i_hbm, ag_hbm):
        def body_ag(i_vmem, o_vmem):
            pltpu.sync_copy(tbl_hbm.at[i_vmem.at[0]], o_vmem)

        pltpu.emit_pipeline(
            body_ag,
            grid=(N_E16 // _GW,),
            in_specs=[pl.BlockSpec((1, _GW), index_map=lambda i: (0, i))],
            out_specs=[pl.BlockSpec((_GW, 3 * C), index_map=lambda i: (i, 0))],
            core_axis_name=("c", "s"),
            dimension_semantics=(pltpu.PARALLEL,),
        )(i_hbm, ag_hbm)

    return gather_kernel(table, idx_flat.reshape(1, N_E16))


def kernel(x1, pos1, x2, pos2, W_l1, b_l1, W_l2, b_l2, W_val, W_src, W_dst,
           Wp1, bp1, gp, btp, Wp2, bp2, Wa1, ba1, ga, bta, Wa2, ba2,
           W_up, b_up):
    f32 = jnp.float32

    def q_of(pos, n_pad):
        q = jnp.zeros((pos.shape[0], 8), f32).at[:, :3].set(pos).at[:, 3].set(1.0)
        return q

    def pt_of(pos, n_pad):
        n = pos.shape[0]
        pt = jnp.zeros((8, n_pad), f32)
        pt = pt.at[:3, :n].set(pos.T)
        pt = pt.at[3, n:].set(_BIG)
        return pt

    x1p = jnp.pad(x1, ((0, N1P - N1), (0, 0)))
    pos2q = q_of(pos2, None)
    wp1q = jnp.zeros((C, 8), f32).at[:, :3].set(Wp1)
    x_int, v, a_src, a_dst, u = _interp_stage(
        x1p, pt_of(pos1, N1P), pos2q, W_l1, b_l1.reshape(1, C),
        W_val, W_src, W_dst, wp1q)
    idx = _knn16_stage(pt_of(pos2, N2P), pos2q)
    table = jnp.concatenate([a_src, v, u], axis=1)
    ag = _sc_gather(table, idx)
    row = lambda x: x.reshape(1, C).astype(f32)
    out = _edge_stage(ag, a_dst, a_src, v, x_int, u,
                      row(bp1), row(gp), row(btp), Wp2, row(bp2),
                      Wa1, row(ba1), row(ga), row(bta), Wa2, row(ba2),
                      W_up, row(b_up))
    return out


# speedup vs baseline: 1.1081x; 1.1081x over previous
import functools

import jax
import jax.numpy as jnp
from jax.experimental import pallas as pl
from jax.experimental.pallas import tpu as pltpu
from jax.experimental.pallas import tpu_sc as plsc

N1, N2, C = 2500, 10000, 128
K_UP, K_G = 8, 16
R = 400
NB = N2 // R
EB = R * K_G
N_EDGE = N2 * (K_G + 1)
_BN_EPS = 1e-5


def _rep16(x):
    r, d = x.shape
    return jnp.broadcast_to(x[:, None, :], (r, K_G, d)).reshape(r * K_G, d)


def _sum8(x):
    n, d = x.shape
    return jnp.sum(x.reshape(n // 8, 8, d), axis=0)


def _edge_kernel(ag_ref, adst_ref, asrc_ref, v_ref, xint_ref, u_ref,
                 bp1_ref, gp_ref, btp_ref, wp2_ref, bp2_ref,
                 wa1_ref, ba1_ref, ga_ref, bta_ref, wa2_ref, ba2_ref,
                 wup_ref, bup_ref, out_ref,
                 sp1, sp2, sa1, sa2):
    p = pl.program_id(0)

    @pl.when((p == 0) & (pl.program_id(1) == 0))
    def _():
        sp1[...] = jnp.zeros_like(sp1)
        sp2[...] = jnp.zeros_like(sp2)
        sa1[...] = jnp.zeros_like(sa1)
        sa2[...] = jnp.zeros_like(sa2)

    f32 = jnp.float32
    dot = functools.partial(jnp.dot, preferred_element_type=f32)

    hp = (_rep16(u_ref[...]) + bp1_ref[...]) - ag_ref[:, 2 * C:]

    @pl.when(p == 0)
    def _():
        sp1[...] += _sum8(hp)
        sp2[...] += _sum8(hp * hp)
        out_ref[...] = jnp.zeros_like(out_ref)

    @pl.when(p > 0)
    def _():
        bp1v = bp1_ref[...]
        n_e = f32(N_EDGE)
        mu_p = (jnp.sum(sp1[...], 0, keepdims=True) + N2 * bp1v) / n_e
        msq_p = (jnp.sum(sp2[...], 0, keepdims=True) + N2 * bp1v * bp1v) / n_e
        inv_p = jax.lax.rsqrt(msq_p - mu_p * mu_p + _BN_EPS)

        def bn_p(h):
            return (h - mu_p) * inv_p * gp_ref[...] + btp_ref[...]

        delta = dot(jax.nn.relu(bn_p(hp)), wp2_ref[...].T) + bp2_ref[...]
        d0 = dot(jax.nn.relu(bn_p(bp1v)), wp2_ref[...].T) + bp2_ref[...]

        adst = adst_ref[...]
        alpha_in = _rep16(adst) - ag_ref[:, :C] + delta
        ha = dot(alpha_in, wa1_ref[...].T) + ba1_ref[...]
        ha_s = dot(adst - asrc_ref[...] + d0, wa1_ref[...].T) + ba1_ref[...]

        @pl.when(p == 1)
        def _():
            sa1[...] += _sum8(ha) + _sum8(ha_s)
            sa2[...] += _sum8(ha * ha) + _sum8(ha_s * ha_s)
            out_ref[...] = jnp.zeros_like(out_ref)

        @pl.when(p == 2)
        def _():
            mu_a = jnp.sum(sa1[...], 0, keepdims=True) / n_e
            msq_a = jnp.sum(sa2[...], 0, keepdims=True) / n_e
            inv_a = jax.lax.rsqrt(msq_a - mu_a * mu_a + _BN_EPS)

            def bn_a(h):
                return (h - mu_a) * inv_a * ga_ref[...] + bta_ref[...]

            al = dot(jax.nn.relu(bn_a(ha)), wa2_ref[...].T) + ba2_ref[...]
            al_s = dot(jax.nn.relu(bn_a(ha_s)), wa2_ref[...].T) + ba2_ref[...]

            al3 = al.reshape(R, K_G, C)
            m = jnp.maximum(jnp.max(al3, axis=1), al_s)
            ex = jnp.exp(al3 - m[:, None, :])
            ex_s = jnp.exp(al_s - m)
            den = jnp.sum(ex, axis=1) + ex_s + 1e-16
            vpd = (ag_ref[:, C:2 * C] + delta).reshape(R, K_G, C)
            num = jnp.sum(ex * vpd, axis=1) + ex_s * (v_ref[...] + d0)
            agg = num / den
            out_ref[...] = (dot(agg, wup_ref[...].T) + bup_ref[...]
                            + xint_ref[...])


def _edge_stage(ag, a_dst, a_src, v, x_int, u, bp1, gp, btp,
                wp2, bp2, wa1, ba1, ga, bta, wa2, ba2, w_up, b_up):
    blk = lambda r, c: pl.BlockSpec((r, c), lambda p, b: (b, 0))
    full = lambda r, c: pl.BlockSpec((r, c), lambda p, b: (0, 0))
    return pl.pallas_call(
        _edge_kernel,
        grid=(3, NB),
        in_specs=[
            blk(EB, 3 * C),
            blk(R, C), blk(R, C), blk(R, C), blk(R, C), blk(R, C),
            full(1, C), full(1, C), full(1, C),
            full(C, C), full(1, C),
            full(C, C), full(1, C), full(1, C), full(1, C),
            full(C, C), full(1, C),
            full(C, C), full(1, C),
        ],
        out_specs=blk(R, C),
        out_shape=jax.ShapeDtypeStruct((N2, C), jnp.float32),
        scratch_shapes=[pltpu.VMEM((8, C), jnp.float32)] * 4,
    )(ag, a_dst, a_src, v, x_int, u, bp1, gp, btp,
      wp2, bp2, wa1, ba1, ga, bta, wa2, ba2, w_up, b_up)


_BIG = 1e30
N1P = 2560
N2P = 10240


def _interp_kernel(x1_ref, pt1_ref, q_ref, wl1_ref, bl1_ref,
                   wv_ref, ws_ref, wd_ref, wp1q_ref,
                   xint_ref, v_ref, asrc_ref, adst_ref, u_ref, h1_ref):
    f32 = jnp.float32
    dot = functools.partial(jnp.dot, preferred_element_type=f32,
                            precision=jax.lax.Precision.HIGHEST)

    @pl.when(pl.program_id(0) == 0)
    def _():
        h1_ref[...] = dot(x1_ref[...], wl1_ref[...].T) + bl1_ref[...]

    d2 = _dist2(q_ref, pt1_ref)
    score = d2
    sel = jnp.zeros_like(score, dtype=jnp.bool_)
    cols = jax.lax.broadcasted_iota(jnp.int32, (1, N1P), 1)
    for _ in range(K_UP):
        val = jnp.min(score, axis=1, keepdims=True)
        eq = score == val
        idxc = jnp.where(eq, cols, jnp.int32(2 ** 30))
        si = jnp.min(idxc, axis=1, keepdims=True)
        hit = cols == si
        sel = sel | hit
        score = jnp.where(hit, _BIG, score)
    s = jnp.where(sel, 1.0 / jnp.clip(d2, 1e-16, None), 0.0)
    x_int = dot(s, h1_ref[...]) / jnp.sum(s, axis=1, keepdims=True)
    xint_ref[...] = x_int
    v_ref[...] = dot(x_int, wv_ref[...].T)
    asrc_ref[...] = dot(x_int, ws_ref[...].T)
    adst_ref[...] = dot(x_int, wd_ref[...].T)
    u_ref[...] = dot(q_ref[...], wp1q_ref[...].T)


def _interp_stage(x1p, pt1, pos2q, W_l1, b_l1, W_val, W_src, W_dst, Wp1q):
    blk = pl.BlockSpec((R, C), lambda b: (b, 0))
    full = lambda r, c: pl.BlockSpec((r, c), lambda b: (0, 0))
    o = jax.ShapeDtypeStruct((N2, C), jnp.float32)
    return pl.pallas_call(
        _interp_kernel,
        grid=(NB,),
        in_specs=[full(N1P, C), full(8, N1P), pl.BlockSpec((R, 8), lambda b: (b, 0)),
                  full(C, C), full(1, C), full(C, C), full(C, C), full(C, C),
                  full(C, 8)],
        out_specs=[blk] * 5,
        out_shape=[o, o, o, o, o],
        scratch_shapes=[pltpu.VMEM((N1P, C), jnp.float32)],
    )(x1p, pt1, pos2q, W_l1, b_l1, W_val, W_src, W_dst, Wp1q)


def _dist2(q_ref, pt_ref):
    dx = q_ref[:, 0:1] - pt_ref[0:1, :]
    dy = q_ref[:, 1:2] - pt_ref[1:2, :]
    dz = q_ref[:, 2:3] - pt_ref[2:3, :]
    return (dx * dx + dy * dy) + dz * dz + pt_ref[3:4, :]


def _knn16_kernel(pt2_ref, q_ref, idx_ref):
    b = pl.program_id(0)
    score = _dist2(q_ref, pt2_ref)
    cols = jax.lax.broadcasted_iota(jnp.int32, (1, N2P), 1)
    rows = jax.lax.broadcasted_iota(jnp.int32, (R, 1), 0)
    score = jnp.where(cols == b * R + rows, _BIG, score)
    lane16 = jax.lax.broadcasted_iota(jnp.int32, (1, K_G), 1)
    idx_out = jnp.zeros((R, K_G), jnp.int32)
    for k in range(K_G):
        val = jnp.min(score, axis=1, keepdims=True)
        eq = score == val
        idxc = jnp.where(eq, cols, jnp.int32(2 ** 30))
        si = jnp.min(idxc, axis=1, keepdims=True)
        score = jnp.where(cols == si, _BIG, score)
        idx_out = jnp.where(lane16 == k, si, idx_out)
    idx_ref[...] = idx_out


def _knn16_stage(pt2, pos2q):
    return pl.pallas_call(
        _knn16_kernel,
        grid=(NB,),
        in_specs=[pl.BlockSpec((8, N2P), lambda b: (0, 0)),
                  pl.BlockSpec((R, 8), lambda b: (b, 0))],
        out_specs=pl.BlockSpec((R, K_G), lambda b: (b, 0)),
        out_shape=jax.ShapeDtypeStruct((N2, K_G), jnp.int32),
    )(pt2, pos2q)


_GW = 128
N_E16 = N2 * K_G


def _sc_gather(table, idx_flat):
    f32 = jnp.float32
    mesh = plsc.VectorSubcoreMesh(core_axis_name="c", subcore_axis_name="s")

    @functools.partial(
        pl.kernel,
        out_type=jax.ShapeDtypeStruct((N_E16, 3 * C), f32),
        mesh=mesh)
    def gather_kernel(tbl_hbm, i_hbm, ag_hbm):
        def body_ag(i_vmem, o_vmem):
            pltpu.sync_copy(tbl_hbm.at[i_vmem.at[0]], o_vmem)

        pltpu.emit_pipeline(
            body_ag,
            grid=(N_E16 // _GW,),
            in_specs=[pl.BlockSpec((1, _GW), index_map=lambda i: (0, i))],
            out_specs=[pl.BlockSpec((_GW, 3 * C), index_map=lambda i: (i, 0))],
            core_axis_name=("c", "s"),
            dimension_semantics=(pltpu.PARALLEL,),
        )(i_hbm, ag_hbm)

    return gather_kernel(table, idx_flat.reshape(1, N_E16))


def kernel(x1, pos1, x2, pos2, W_l1, b_l1, W_l2, b_l2, W_val, W_src, W_dst,
           Wp1, bp1, gp, btp, Wp2, bp2, Wa1, ba1, ga, bta, Wa2, ba2,
           W_up, b_up):
    f32 = jnp.float32

    def q_of(pos, n_pad):
        q = jnp.zeros((pos.shape[0], 8), f32).at[:, :3].set(pos).at[:, 3].set(1.0)
        return q

    def pt_of(pos, n_pad):
        n = pos.shape[0]
        pt = jnp.zeros((8, n_pad), f32)
        pt = pt.at[:3, :n].set(pos.T)
        pt = pt.at[3, n:].set(_BIG)
        return pt

    x1p = jnp.pad(x1, ((0, N1P - N1), (0, 0)))
    pos2q = q_of(pos2, None)
    wp1q = jnp.zeros((C, 8), f32).at[:, :3].set(Wp1)
    x_int, v, a_src, a_dst, u = _interp_stage(
        x1p, pt_of(pos1, N1P), pos2q, W_l1, b_l1.reshape(1, C),
        W_val, W_src, W_dst, wp1q)
    idx = _knn16_stage(pt_of(pos2, N2P), pos2q)
    table = jnp.concatenate([a_src, v, u], axis=1)
    ag = _sc_gather(table, idx)
    row = lambda x: x.reshape(1, C).astype(f32)
    out = _edge_stage(ag, a_dst, a_src, v, x_int, u,
                      row(bp1), row(gp), row(btp), Wp2, row(bp2),
                      Wa1, row(ba1), row(ga), row(bta), Wa2, row(ba2),
                      W_up, row(b_up))
    return out
